# baseline (speedup 1.0000x reference)
import functools

import jax
import jax.numpy as jnp
from jax import lax
from jax.experimental import pallas as pl
from jax.experimental.pallas import tpu as pltpu
from jax.experimental.pallas import tpu_sc as plsc

N_CORES = 2
N_SUBCORES = 16
N_WORKERS = N_CORES * N_SUBCORES
CHUNK = 64
LANES = 16
NBUF = 4
LOOK = NBUF - 2
WCH = 8
WSLOTS = 4


def _linear(x, wt, b2):
    M, K = x.shape
    N = wt.shape[1]
    BM = 1000

    def body(x_ref, w_ref, b_ref, o_ref):
        o_ref[...] = (
            jnp.dot(x_ref[...], w_ref[...], preferred_element_type=jnp.float32)
            + b_ref[...]
        )

    return pl.pallas_call(
        body,
        grid=(M // BM,),
        in_specs=[
            pl.BlockSpec((BM, K), lambda i: (i, 0)),
            pl.BlockSpec((K, N), lambda i: (0, 0)),
            pl.BlockSpec((1, N), lambda i: (0, 0)),
        ],
        out_specs=pl.BlockSpec((BM, N), lambda i: (i, 0)),
        out_shape=jax.ShapeDtypeStruct((M, N), jnp.float32),
    )(x, wt, b2)


def _combine(p0, p1):
    M, N = p0.shape
    BM = 1000

    def body(a_ref, b_ref, o_ref):
        o_ref[...] = a_ref[...] + b_ref[...]

    return pl.pallas_call(
        body,
        grid=(M // BM,),
        in_specs=[
            pl.BlockSpec((BM, N), lambda i: (i, 0)),
            pl.BlockSpec((BM, N), lambda i: (i, 0)),
        ],
        out_specs=pl.BlockSpec((BM, N), lambda i: (i, 0)),
        out_shape=jax.ShapeDtypeStruct((M, N), jnp.float32),
    )(p0, p1)


@functools.lru_cache(maxsize=None)
def _make_spmm(n_nodes, d, c0pw, c1pw):
    acc_rows = 10240
    rows_per_tile = acc_rows // N_SUBCORES
    zrows = CHUNK
    mesh = plsc.VectorSubcoreMesh(core_axis_name="c", subcore_axis_name="s")

    @functools.partial(
        pl.kernel,
        mesh=mesh,
        out_type=jax.ShapeDtypeStruct((N_CORES, acc_rows, d), jnp.float32),
        scratch_types=[
            pltpu.VMEM_SHARED((acc_rows, d), jnp.float32),
            pltpu.VMEM((WSLOTS * WCH, CHUNK), jnp.int32),
            pltpu.VMEM((WSLOTS * WCH, CHUNK), jnp.int32),
            pltpu.VMEM((WSLOTS * WCH, CHUNK), jnp.float32),
            pltpu.VMEM((CHUNK, d), jnp.float32),
            pltpu.VMEM((CHUNK, d), jnp.float32),
            pltpu.VMEM((CHUNK, d), jnp.float32),
            pltpu.VMEM((CHUNK, d), jnp.float32),
            pltpu.SemaphoreType.DMA,
            pltpu.SemaphoreType.DMA,
            pltpu.SemaphoreType.DMA,
        ],
    )
    def spmm(h_hbm, src_hbm, dst_hbm, attr_hbm, out_hbm,
             acc, src_w, dst_w, attr_w, rb0, rb1, rb2, rb3,
             gsem, ssem, isem):
        rbufs = (rb0, rb1, rb2, rb3)
        c = lax.axis_index("c")
        s = lax.axis_index("s")
        base = jnp.where(c == 0, s * c0pw, N_SUBCORES * c0pw + s * c1pw)
        n = jnp.where(c == 0, c0pw, c1pw)
        nwin = n // WCH

        def load_window(w):
            row = (w % WSLOTS) * WCH
            wb = base + w * WCH
            pltpu.async_copy(src_hbm.at[pl.ds(wb, WCH)],
                             src_w.at[pl.ds(row, WCH)], isem)
            pltpu.async_copy(dst_hbm.at[pl.ds(wb, WCH)],
                             dst_w.at[pl.ds(row, WCH)], isem)
            pltpu.async_copy(attr_hbm.at[pl.ds(wb, WCH)],
                             attr_w.at[pl.ds(row, WCH)], isem)

        def wait_window():
            for _ in range(3):
                pltpu.make_async_copy(
                    src_hbm.at[pl.ds(0, WCH)], src_w.at[pl.ds(0, WCH)], isem
                ).wait()

        def wait_gather():
            pltpu.make_async_copy(h_hbm.at[pl.ds(0, CHUNK)], rb0, gsem).wait()

        def wait_scatter():
            pltpu.make_async_copy(rb0, acc.at[pl.ds(0, CHUNK)], ssem).wait()

        def issue_gather(j, buf):
            pltpu.async_copy(h_hbm.at[src_w.at[j % (WSLOTS * WCH)]], buf, gsem)

        def issue_scatter(j, buf):
            pltpu.async_copy(buf, acc.at[dst_w.at[j % (WSLOTS * WCH)]], ssem,
                             add=True)

        zv = jnp.zeros((LANES,), jnp.float32)

        def zrow(i, carry):
            for q in range(d // LANES):
                rb0[i, pl.ds(q * LANES, LANES)] = zv
            return carry

        lax.fori_loop(0, zrows, zrow, 0)

        def zacc(k, carry):
            pltpu.sync_copy(rb0, acc.at[pl.ds(s * rows_per_tile + k * zrows, zrows)])
            return carry

        lax.fori_loop(0, rows_per_tile // zrows, zacc, 0)
        plsc.subcore_barrier()

        load_window(0)
        wait_window()

        @pl.when(nwin > 1)
        def _():
            load_window(1)

        @pl.when(nwin > 2)
        def _():
            load_window(2)
        issue_gather(0, rbufs[0])
        issue_gather(1, rbufs[1])

        def quad_body(jj, carry):
            for p in range(NBUF):
                j = jj * NBUF + p
                jg = j + LOOK

                @pl.when(jnp.logical_and(j % WCH == 2, j // WCH + 3 < nwin))
                def _():
                    load_window(j // WCH + 3)

                @pl.when(jg < n)
                def _():
                    @pl.when(jg % WCH == 0)
                    def _():
                        wait_window()

                    @pl.when(j >= 2)
                    def _():
                        wait_scatter()

                    issue_gather(jg, rbufs[(p + LOOK) % NBUF])

                wait_gather()
                rbuf = rbufs[p]

                def group_body(g, carry2):
                    av = attr_w[j % (WSLOTS * WCH), pl.ds(g * LANES, LANES)]
                    for i in range(LANES):
                        a = av[i]
                        e = g * LANES + i
                        for q in range(d // LANES):
                            rbuf[e, pl.ds(q * LANES, LANES)] = (
                                rbuf[e, pl.ds(q * LANES, LANES)] * a
                            )
                    return carry2

                lax.fori_loop(0, CHUNK // LANES, group_body, 0)
                issue_scatter(j, rbuf)
            return carry

        @pl.when(c == 0)
        def _():
            lax.fori_loop(0, c0pw // NBUF, quad_body, 0)

        @pl.when(c == 1)
        def _():
            lax.fori_loop(0, c1pw // NBUF, quad_body, 0)

        for _ in range(NBUF):
            wait_scatter()
        plsc.subcore_barrier()

        pltpu.sync_copy(
            acc.at[pl.ds(s * rows_per_tile, rows_per_tile)],
            out_hbm.at[c, pl.ds(s * rows_per_tile, rows_per_tile)],
        )

    return spmm


def kernel(x, edge_index, edge_attr, W, b):
    n_nodes, d = x.shape
    n_edges = edge_attr.shape[0]
    per_sc = -(-n_edges // (N_SUBCORES * CHUNK * 16)) * 16
    total_chunks = per_sc * N_SUBCORES
    c0pw = min((per_sc * 10 // 10 // 8) * 8, per_sc - 8)
    c1pw = per_sc - c0pw
    e_pad = total_chunks * CHUNK

    dst = jnp.pad(edge_index[0], (0, e_pad - n_edges)).reshape(-1, CHUNK)
    src = jnp.pad(edge_index[1], (0, e_pad - n_edges)).reshape(-1, CHUNK)
    attr = jnp.pad(edge_attr, (0, e_pad - n_edges)).reshape(-1, CHUNK)

    h = _linear(x, W.T, b.reshape(1, -1))
    spmm = _make_spmm(n_nodes, d, c0pw, c1pw)
    for _ in range(3):
        partials = spmm(h, src, dst, attr)
        h = _combine(partials[0, :n_nodes], partials[1, :n_nodes])
    return h


# speedup vs baseline: 1.0119x; 1.0119x over previous
import functools

import jax
import jax.numpy as jnp
from jax import lax
from jax.experimental import pallas as pl
from jax.experimental.pallas import tpu as pltpu
from jax.experimental.pallas import tpu_sc as plsc

N_CORES = 2
N_SUBCORES = 16
N_WORKERS = N_CORES * N_SUBCORES
CHUNK = 64
LANES = 16
NBUF = 4
LOOK = NBUF - 2
WCH = 8
WSLOTS = 4


def _linear(x, wt, b2):
    M, K = x.shape
    N = wt.shape[1]
    BM = 1000

    def body(x_ref, w_ref, b_ref, o_ref):
        o_ref[...] = (
            jnp.dot(x_ref[...], w_ref[...], preferred_element_type=jnp.float32)
            + b_ref[...]
        )

    return pl.pallas_call(
        body,
        grid=(M // BM,),
        in_specs=[
            pl.BlockSpec((BM, K), lambda i: (i, 0)),
            pl.BlockSpec((K, N), lambda i: (0, 0)),
            pl.BlockSpec((1, N), lambda i: (0, 0)),
        ],
        out_specs=pl.BlockSpec((BM, N), lambda i: (i, 0)),
        out_shape=jax.ShapeDtypeStruct((M, N), jnp.float32),
    )(x, wt, b2)


def _combine(p0, p1):
    M, N = p0.shape
    BM = 1000

    def body(a_ref, b_ref, o_ref):
        o_ref[...] = a_ref[...] + b_ref[...]

    return pl.pallas_call(
        body,
        grid=(M // BM,),
        in_specs=[
            pl.BlockSpec((BM, N), lambda i: (i, 0)),
            pl.BlockSpec((BM, N), lambda i: (i, 0)),
        ],
        out_specs=pl.BlockSpec((BM, N), lambda i: (i, 0)),
        out_shape=jax.ShapeDtypeStruct((M, N), jnp.float32),
    )(p0, p1)


@functools.lru_cache(maxsize=None)
def _make_spmm(n_nodes, d, c0pw, c1pw):
    acc_rows = 10240
    rows_per_tile = acc_rows // N_SUBCORES
    zrows = CHUNK
    mesh = plsc.VectorSubcoreMesh(core_axis_name="c", subcore_axis_name="s")

    @functools.partial(
        pl.kernel,
        mesh=mesh,
        out_type=jax.ShapeDtypeStruct((N_CORES, acc_rows, d), jnp.float32),
        scratch_types=[
            pltpu.VMEM_SHARED((acc_rows, d), jnp.float32),
            pltpu.VMEM((WSLOTS * WCH, CHUNK), jnp.int32),
            pltpu.VMEM((WSLOTS * WCH, CHUNK), jnp.int32),
            pltpu.VMEM((WSLOTS * WCH, CHUNK), jnp.float32),
            pltpu.VMEM((CHUNK, d), jnp.float32),
            pltpu.VMEM((CHUNK, d), jnp.float32),
            pltpu.VMEM((CHUNK, d), jnp.float32),
            pltpu.VMEM((CHUNK, d), jnp.float32),
            pltpu.SemaphoreType.DMA,
            pltpu.SemaphoreType.DMA,
            pltpu.SemaphoreType.DMA,
        ],
    )
    def spmm(h_hbm, src_hbm, dst_hbm, attr_hbm, out_hbm,
             acc, src_w, dst_w, attr_w, rb0, rb1, rb2, rb3,
             gsem, ssem, isem):
        rbufs = (rb0, rb1, rb2, rb3)
        c = lax.axis_index("c")
        s = lax.axis_index("s")
        base = jnp.where(c == 0, s * c0pw, N_SUBCORES * c0pw + s * c1pw)
        n = jnp.where(c == 0, c0pw, c1pw)
        nwin = n // WCH

        def load_window(w):
            row = (w % WSLOTS) * WCH
            wb = base + w * WCH
            pltpu.async_copy(src_hbm.at[pl.ds(wb, WCH)],
                             src_w.at[pl.ds(row, WCH)], isem)
            pltpu.async_copy(dst_hbm.at[pl.ds(wb, WCH)],
                             dst_w.at[pl.ds(row, WCH)], isem)
            pltpu.async_copy(attr_hbm.at[pl.ds(wb, WCH)],
                             attr_w.at[pl.ds(row, WCH)], isem)

        def wait_window():
            for _ in range(3):
                pltpu.make_async_copy(
                    src_hbm.at[pl.ds(0, WCH)], src_w.at[pl.ds(0, WCH)], isem
                ).wait()

        def wait_gather():
            pltpu.make_async_copy(h_hbm.at[pl.ds(0, CHUNK)], rb0, gsem).wait()

        def wait_scatter():
            pltpu.make_async_copy(rb0, acc.at[pl.ds(0, CHUNK)], ssem).wait()

        def issue_gather(j, buf):
            pltpu.async_copy(h_hbm.at[src_w.at[j % (WSLOTS * WCH)]], buf, gsem)

        def issue_scatter(j, buf):
            pltpu.async_copy(buf, acc.at[dst_w.at[j % (WSLOTS * WCH)]], ssem,
                             add=True)

        zv = jnp.zeros((LANES,), jnp.float32)

        def zrow(i, carry):
            for q in range(d // LANES):
                rb0[i, pl.ds(q * LANES, LANES)] = zv
            return carry

        lax.fori_loop(0, zrows, zrow, 0)

        def zacc(k, carry):
            pltpu.sync_copy(rb0, acc.at[pl.ds(s * rows_per_tile + k * zrows, zrows)])
            return carry

        lax.fori_loop(0, rows_per_tile // zrows, zacc, 0)
        plsc.subcore_barrier()

        load_window(0)
        wait_window()

        @pl.when(nwin > 1)
        def _():
            load_window(1)

        @pl.when(nwin > 2)
        def _():
            load_window(2)
        issue_gather(0, rbufs[0])
        issue_gather(1, rbufs[1])

        def quad_body(jj, carry):
            for p in range(NBUF):
                j = jj * NBUF + p
                jg = j + LOOK

                @pl.when(jnp.logical_and(j % WCH == 2, j // WCH + 3 < nwin))
                def _():
                    load_window(j // WCH + 3)

                @pl.when(jg < n)
                def _():
                    @pl.when(jg % WCH == 0)
                    def _():
                        wait_window()

                    @pl.when(j >= 2)
                    def _():
                        wait_scatter()

                    issue_gather(jg, rbufs[(p + LOOK) % NBUF])

                wait_gather()
                rbuf = rbufs[p]

                def group_body(g, carry2):
                    av = attr_w[j % (WSLOTS * WCH), pl.ds(g * LANES, LANES)]
                    for i in range(LANES):
                        a = av[i]
                        e = g * LANES + i
                        for q in range(d // LANES):
                            rbuf[e, pl.ds(q * LANES, LANES)] = (
                                rbuf[e, pl.ds(q * LANES, LANES)] * a
                            )
                    return carry2

                lax.fori_loop(0, CHUNK // LANES, group_body, 0)
                issue_scatter(j, rbuf)
            return carry

        @pl.when(c == 0)
        def _():
            lax.fori_loop(0, c0pw // NBUF, quad_body, 0)

        @pl.when(c == 1)
        def _():
            lax.fori_loop(0, c1pw // NBUF, quad_body, 0)

        for _ in range(NBUF):
            wait_scatter()
        plsc.subcore_barrier()

        pltpu.sync_copy(
            acc.at[pl.ds(s * rows_per_tile, rows_per_tile)],
            out_hbm.at[c, pl.ds(s * rows_per_tile, rows_per_tile)],
        )

    return spmm


def kernel(x, edge_index, edge_attr, W, b):
    n_nodes, d = x.shape
    n_edges = edge_attr.shape[0]
    per_sc = -(-n_edges // (N_SUBCORES * CHUNK * 16)) * 16
    total_chunks = per_sc * N_SUBCORES
    c0pw = min((per_sc * 9 // 10 // 8) * 8, per_sc - 8)
    c1pw = per_sc - c0pw
    e_pad = total_chunks * CHUNK

    dst = jnp.pad(edge_index[0], (0, e_pad - n_edges)).reshape(-1, CHUNK)
    src = jnp.pad(edge_index[1], (0, e_pad - n_edges)).reshape(-1, CHUNK)
    attr = jnp.pad(edge_attr, (0, e_pad - n_edges)).reshape(-1, CHUNK)

    h = _linear(x, W.T, b.reshape(1, -1))
    spmm = _make_spmm(n_nodes, d, c0pw, c1pw)
    for _ in range(3):
        partials = spmm(h, src, dst, attr)
        h = _combine(partials[0, :n_nodes], partials[1, :n_nodes])
    return h


# speedup vs baseline: 1.0135x; 1.0016x over previous
import functools

import jax
import jax.numpy as jnp
from jax import lax
from jax.experimental import pallas as pl
from jax.experimental.pallas import tpu as pltpu
from jax.experimental.pallas import tpu_sc as plsc

N_CORES = 2
N_SUBCORES = 16
N_WORKERS = N_CORES * N_SUBCORES
CHUNK = 64
LANES = 16
NBUF = 4
LOOK = NBUF - 2
WCH = 8
WSLOTS = 4


def _linear(x, wt, b2):
    M, K = x.shape
    N = wt.shape[1]
    BM = 1000

    def body(x_ref, w_ref, b_ref, o_ref):
        o_ref[...] = (
            jnp.dot(x_ref[...], w_ref[...], preferred_element_type=jnp.float32)
            + b_ref[...]
        )

    return pl.pallas_call(
        body,
        grid=(M // BM,),
        in_specs=[
            pl.BlockSpec((BM, K), lambda i: (i, 0)),
            pl.BlockSpec((K, N), lambda i: (0, 0)),
            pl.BlockSpec((1, N), lambda i: (0, 0)),
        ],
        out_specs=pl.BlockSpec((BM, N), lambda i: (i, 0)),
        out_shape=jax.ShapeDtypeStruct((M, N), jnp.float32),
    )(x, wt, b2)


def _combine(p0, p1):
    M, N = p0.shape
    BM = 1000

    def body(a_ref, b_ref, o_ref):
        o_ref[...] = a_ref[...] + b_ref[...]

    return pl.pallas_call(
        body,
        grid=(M // BM,),
        in_specs=[
            pl.BlockSpec((BM, N), lambda i: (i, 0)),
            pl.BlockSpec((BM, N), lambda i: (i, 0)),
        ],
        out_specs=pl.BlockSpec((BM, N), lambda i: (i, 0)),
        out_shape=jax.ShapeDtypeStruct((M, N), jnp.float32),
    )(p0, p1)


@functools.lru_cache(maxsize=None)
def _make_spmm(n_nodes, d, c0pw, c1pw):
    acc_rows = 10240
    rows_per_tile = acc_rows // N_SUBCORES
    zrows = CHUNK
    mesh = plsc.VectorSubcoreMesh(core_axis_name="c", subcore_axis_name="s")

    @functools.partial(
        pl.kernel,
        mesh=mesh,
        out_type=jax.ShapeDtypeStruct((N_CORES, acc_rows, d), jnp.float32),
        scratch_types=[
            pltpu.VMEM_SHARED((acc_rows, d), jnp.float32),
            pltpu.VMEM((WSLOTS * WCH, CHUNK), jnp.int32),
            pltpu.VMEM((WSLOTS * WCH, CHUNK), jnp.int32),
            pltpu.VMEM((WSLOTS * WCH, CHUNK), jnp.float32),
            pltpu.VMEM((CHUNK, d), jnp.float32),
            pltpu.VMEM((CHUNK, d), jnp.float32),
            pltpu.VMEM((CHUNK, d), jnp.float32),
            pltpu.VMEM((CHUNK, d), jnp.float32),
            pltpu.SemaphoreType.DMA,
            pltpu.SemaphoreType.DMA,
            pltpu.SemaphoreType.DMA,
        ],
    )
    def spmm(h_hbm, src_hbm, dst_hbm, attr_hbm, out_hbm,
             acc, src_w, dst_w, attr_w, rb0, rb1, rb2, rb3,
             gsem, ssem, isem):
        rbufs = (rb0, rb1, rb2, rb3)
        c = lax.axis_index("c")
        s = lax.axis_index("s")
        base = jnp.where(c == 0, s * c0pw, N_SUBCORES * c0pw + s * c1pw)
        n = jnp.where(c == 0, c0pw, c1pw)
        nwin = n // WCH

        def load_window(w):
            row = (w % WSLOTS) * WCH
            wb = base + w * WCH
            pltpu.async_copy(src_hbm.at[pl.ds(wb, WCH)],
                             src_w.at[pl.ds(row, WCH)], isem)
            pltpu.async_copy(dst_hbm.at[pl.ds(wb, WCH)],
                             dst_w.at[pl.ds(row, WCH)], isem)
            pltpu.async_copy(attr_hbm.at[pl.ds(wb, WCH)],
                             attr_w.at[pl.ds(row, WCH)], isem)

        def wait_window():
            for _ in range(3):
                pltpu.make_async_copy(
                    src_hbm.at[pl.ds(0, WCH)], src_w.at[pl.ds(0, WCH)], isem
                ).wait()

        def wait_gather():
            pltpu.make_async_copy(h_hbm.at[pl.ds(0, CHUNK)], rb0, gsem).wait()

        def wait_scatter():
            pltpu.make_async_copy(rb0, acc.at[pl.ds(0, CHUNK)], ssem).wait()

        def issue_gather(j, buf):
            pltpu.async_copy(h_hbm.at[src_w.at[j % (WSLOTS * WCH)]], buf, gsem)

        def issue_scatter(j, buf):
            pltpu.async_copy(buf, acc.at[dst_w.at[j % (WSLOTS * WCH)]], ssem,
                             add=True)

        zv = jnp.zeros((LANES,), jnp.float32)

        def zrow(i, carry):
            for q in range(d // LANES):
                rb0[i, pl.ds(q * LANES, LANES)] = zv
            return carry

        lax.fori_loop(0, zrows, zrow, 0)

        def zacc(k, carry):
            pltpu.async_copy(
                rb0, acc.at[pl.ds(s * rows_per_tile + k * zrows, zrows)], ssem
            )
            return carry

        lax.fori_loop(0, rows_per_tile // zrows, zacc, 0)
        for _ in range(rows_per_tile // zrows):
            pltpu.make_async_copy(
                rb0, acc.at[pl.ds(0, zrows)], ssem
            ).wait()
        plsc.subcore_barrier()

        load_window(0)
        wait_window()

        @pl.when(nwin > 1)
        def _():
            load_window(1)

        @pl.when(nwin > 2)
        def _():
            load_window(2)
        issue_gather(0, rbufs[0])
        issue_gather(1, rbufs[1])

        def quad_body(jj, carry):
            for p in range(NBUF):
                j = jj * NBUF + p
                jg = j + LOOK

                @pl.when(jnp.logical_and(j % WCH == 2, j // WCH + 3 < nwin))
                def _():
                    load_window(j // WCH + 3)

                @pl.when(jg < n)
                def _():
                    @pl.when(jg % WCH == 0)
                    def _():
                        wait_window()

                    @pl.when(j >= 2)
                    def _():
                        wait_scatter()

                    issue_gather(jg, rbufs[(p + LOOK) % NBUF])

                wait_gather()
                rbuf = rbufs[p]

                def group_body(g, carry2):
                    av = attr_w[j % (WSLOTS * WCH), pl.ds(g * LANES, LANES)]
                    for i in range(LANES):
                        a = av[i]
                        e = g * LANES + i
                        for q in range(d // LANES):
                            rbuf[e, pl.ds(q * LANES, LANES)] = (
                                rbuf[e, pl.ds(q * LANES, LANES)] * a
                            )
                    return carry2

                lax.fori_loop(0, CHUNK // LANES, group_body, 0)
                issue_scatter(j, rbuf)
            return carry

        @pl.when(c == 0)
        def _():
            lax.fori_loop(0, c0pw // NBUF, quad_body, 0)

        @pl.when(c == 1)
        def _():
            lax.fori_loop(0, c1pw // NBUF, quad_body, 0)

        for _ in range(NBUF):
            wait_scatter()
        plsc.subcore_barrier()

        pltpu.sync_copy(
            acc.at[pl.ds(s * rows_per_tile, rows_per_tile)],
            out_hbm.at[c, pl.ds(s * rows_per_tile, rows_per_tile)],
        )

    return spmm


def kernel(x, edge_index, edge_attr, W, b):
    n_nodes, d = x.shape
    n_edges = edge_attr.shape[0]
    per_sc = -(-n_edges // (N_SUBCORES * CHUNK * 16)) * 16
    total_chunks = per_sc * N_SUBCORES
    c0pw = min((per_sc * 9 // 10 // 8) * 8, per_sc - 8)
    c1pw = per_sc - c0pw
    e_pad = total_chunks * CHUNK

    dst = jnp.pad(edge_index[0], (0, e_pad - n_edges)).reshape(-1, CHUNK)
    src = jnp.pad(edge_index[1], (0, e_pad - n_edges)).reshape(-1, CHUNK)
    attr = jnp.pad(edge_attr, (0, e_pad - n_edges)).reshape(-1, CHUNK)

    h = _linear(x, W.T, b.reshape(1, -1))
    spmm = _make_spmm(n_nodes, d, c0pw, c1pw)
    for _ in range(3):
        partials = spmm(h, src, dst, attr)
        h = _combine(partials[0, :n_nodes], partials[1, :n_nodes])
    return h
